# baseline (speedup 1.0000x reference)
import functools
import math

import jax
import jax.numpy as jnp
import numpy as np
from jax import lax
from jax.experimental import pallas as pl
from jax.experimental.pallas import tpu as pltpu
from jax.experimental.pallas import tpu_sc as plsc

HIDDEN = 2048
HEAD_DIM = 128
ROPE_THETA = 10000.0
_NORM = np.float32(float(HIDDEN) ** 0.5)
_F32_MIN = float(np.finfo(np.float32).min)

_NC = 2
_NS = 16
_NW = _NC * _NS


def _make_sc_gather(n_tokens: int):
    per_w = n_tokens // _NW
    C = 8
    NIN = 4
    NOUT = 2
    nchunks = per_w // C
    mesh = plsc.VectorSubcoreMesh(core_axis_name="c", subcore_axis_name="s")

    @functools.partial(
        pl.kernel,
        mesh=mesh,
        out_type=jax.ShapeDtypeStruct((n_tokens, HIDDEN), jnp.float32),
        scratch_types=(
            [pltpu.VMEM((per_w,), jnp.int32)]
            + [pltpu.VMEM((C, HIDDEN), jnp.float32)] * (NIN + NOUT)
            + [pltpu.SemaphoreType.DMA] * (NIN + NOUT)
        ),
    )
    def sc_gather(ids_hbm, w_hbm, out_hbm, idx_v, *bufs):
        inb = list(bufs[:NIN])
        outb = list(bufs[NIN:NIN + NOUT])
        sin = list(bufs[NIN + NOUT:2 * NIN + NOUT])
        son = list(bufs[2 * NIN + NOUT:])
        wid = lax.axis_index("s") * _NC + lax.axis_index("c")
        base = wid * per_w
        pltpu.sync_copy(ids_hbm.at[pl.ds(base, per_w)], idx_v)

        def gather_copy(c, b):
            return pltpu.make_async_copy(
                w_hbm.at[idx_v.at[pl.ds(c * C, C)]], inb[b], sin[b])

        def out_copy(c, b):
            return pltpu.make_async_copy(
                outb[b], out_hbm.at[pl.ds(base + c * C, C)], son[b])

        for b in range(NIN):
            gather_copy(b, b).start()

        def body(c0):
            for b in range(NIN):
                c = c0 + b
                bo = b % NOUT
                gather_copy(c, b).wait()

                @pl.when(c >= NOUT)
                def _():
                    out_copy(c, bo).wait()

                def scale_row(r, carry):
                    for k in range(HIDDEN // 16):
                        outb[bo][r, pl.ds(k * 16, 16)] = (
                            inb[b][r, pl.ds(k * 16, 16)] * _NORM)
                    return carry

                lax.fori_loop(0, C, scale_row, 0)

                @pl.when(c + NIN < nchunks)
                def _():
                    gather_copy(c + NIN, b).start()

                out_copy(c, bo).start()

        pl.loop(0, nchunks, step=NIN)(body)
        for b in range(NOUT):
            out_copy(nchunks - NOUT + b, b).wait()

    return sc_gather


_BS = 512


def _mask_body(am_ref, out_ref):
    it = pl.program_id(1)
    s = out_ref.shape[2]
    minv = jnp.float32(_F32_MIN)
    padrow = jnp.where(am_ref[0] == 0, minv, jnp.float32(0.0))
    for jt in range(s // _BS):
        colsl = pl.ds(jt * _BS, _BS)
        tile_pad = padrow[:, jt * _BS:(jt + 1) * _BS]

        @pl.when(jt < it)
        def _():
            out_ref[0, :, colsl] = jnp.broadcast_to(tile_pad, (_BS, _BS))

        @pl.when(jt > it)
        def _():
            out_ref[0, :, colsl] = jnp.full((_BS, _BS), minv)

        @pl.when(jt == it)
        def _():
            rows = lax.broadcasted_iota(jnp.int32, (_BS, _BS), 0)
            cols = lax.broadcasted_iota(jnp.int32, (_BS, _BS), 1)
            out_ref[0, :, colsl] = jnp.where(
                cols > rows, minv, jnp.broadcast_to(tile_pad, (_BS, _BS)))


def _make_mask(batch: int, s: int):
    return pl.pallas_call(
        _mask_body,
        grid=(batch, s // _BS),
        in_specs=[pl.BlockSpec((1, 1, s), lambda b, i: (b, 0, 0))],
        out_specs=pl.BlockSpec((1, _BS, s), lambda b, i: (b, i, 0)),
        out_shape=jax.ShapeDtypeStruct((batch, s, s), jnp.float32),
    )


def _rope_body(pos_ref, cos_ref, sin_ref):
    p = pos_ref[:, :].astype(jnp.float32)
    s = p.shape[1]
    k = lax.broadcasted_iota(jnp.int32, (HEAD_DIM, 1), 0)
    k = jnp.where(k >= HEAD_DIM // 2, k - HEAD_DIM // 2, k).astype(jnp.float32)
    inv = jnp.exp(k * jnp.float32(-2.0 * math.log(ROPE_THETA) / HEAD_DIM))
    freqs = inv * p
    cos_ref[:, :] = jnp.cos(freqs)
    sin_ref[:, :] = jnp.sin(freqs)


def _make_rope(s: int):
    return pl.pallas_call(
        _rope_body,
        out_shape=[
            jax.ShapeDtypeStruct((HEAD_DIM, s), jnp.float32),
            jax.ShapeDtypeStruct((HEAD_DIM, s), jnp.float32),
        ],
    )


def kernel(input_ids, attention_mask, position_ids, control_classes, labels, W):
    b, s = input_ids.shape
    ids = input_ids.reshape(-1).astype(jnp.int32)

    mask = _make_mask(b, s)(attention_mask.reshape(b, 1, s)).reshape(b, 1, s, s)

    emb = _make_sc_gather(b * s)(ids, W)
    hidden = emb.reshape(b, s, HIDDEN)

    cos_t, sin_t = _make_rope(s)(position_ids.reshape(1, s))
    cos = cos_t.T.reshape(1, s, HEAD_DIM)
    sin = sin_t.T.reshape(1, s, HEAD_DIM)

    cache_position = jnp.arange(0, s)
    return (hidden, mask, cos, sin, cache_position, control_classes, labels)


# speedup vs baseline: 1.0074x; 1.0074x over previous
import functools
import math

import jax
import jax.numpy as jnp
import numpy as np
from jax import lax
from jax.experimental import pallas as pl
from jax.experimental.pallas import tpu as pltpu
from jax.experimental.pallas import tpu_sc as plsc

HIDDEN = 2048
HEAD_DIM = 128
ROPE_THETA = 10000.0
_NORM = np.float32(float(HIDDEN) ** 0.5)
_F32_MIN = float(np.finfo(np.float32).min)

_NC = 2
_NS = 16
_NW = _NC * _NS


def _make_sc_gather(n_tokens: int):
    per_w = n_tokens // _NW
    C = 8
    NIN = 2
    NOUT = 2
    nchunks = per_w // C
    mesh = plsc.VectorSubcoreMesh(core_axis_name="c", subcore_axis_name="s")

    @functools.partial(
        pl.kernel,
        mesh=mesh,
        out_type=jax.ShapeDtypeStruct((n_tokens, HIDDEN), jnp.float32),
        scratch_types=(
            [pltpu.VMEM((per_w,), jnp.int32)]
            + [pltpu.VMEM((C, HIDDEN), jnp.float32)] * (NIN + NOUT)
            + [pltpu.SemaphoreType.DMA] * (NIN + NOUT)
        ),
    )
    def sc_gather(ids_hbm, w_hbm, out_hbm, idx_v, *bufs):
        inb = list(bufs[:NIN])
        outb = list(bufs[NIN:NIN + NOUT])
        sin = list(bufs[NIN + NOUT:2 * NIN + NOUT])
        son = list(bufs[2 * NIN + NOUT:])
        wid = lax.axis_index("s") * _NC + lax.axis_index("c")
        base = wid * per_w
        pltpu.sync_copy(ids_hbm.at[pl.ds(base, per_w)], idx_v)

        def gather_copy(c, b):
            return pltpu.make_async_copy(
                w_hbm.at[idx_v.at[pl.ds(c * C, C)]], inb[b], sin[b])

        def out_copy(c, b):
            return pltpu.make_async_copy(
                outb[b], out_hbm.at[pl.ds(base + c * C, C)], son[b])

        for b in range(NIN):
            gather_copy(b, b).start()

        def body(c0):
            for b in range(NIN):
                c = c0 + b
                bo = b % NOUT
                gather_copy(c, b).wait()

                @pl.when(c >= NOUT)
                def _():
                    out_copy(c, bo).wait()

                def scale_row(r, carry):
                    for k in range(HIDDEN // 16):
                        outb[bo][r, pl.ds(k * 16, 16)] = (
                            inb[b][r, pl.ds(k * 16, 16)] * _NORM)
                    return carry

                lax.fori_loop(0, C, scale_row, 0)

                @pl.when(c + NIN < nchunks)
                def _():
                    gather_copy(c + NIN, b).start()

                out_copy(c, bo).start()

        pl.loop(0, nchunks, step=NIN)(body)
        for b in range(NOUT):
            out_copy(nchunks - NOUT + b, b).wait()

    return sc_gather


_BS = 1024


def _mask_body(am_ref, out_ref):
    i = pl.program_id(1)
    bs, s = out_ref.shape[1], out_ref.shape[2]
    rows = lax.broadcasted_iota(jnp.int32, (bs, s), 0) + i * bs
    cols = lax.broadcasted_iota(jnp.int32, (bs, s), 1)
    masked = (cols > rows) | (am_ref[0] == 0)
    out_ref[0] = jnp.where(masked, jnp.float32(_F32_MIN), jnp.float32(0.0))


def _make_mask(batch: int, s: int):
    return pl.pallas_call(
        _mask_body,
        grid=(batch, s // _BS),
        in_specs=[pl.BlockSpec((1, 1, s), lambda b, i: (b, 0, 0))],
        out_specs=pl.BlockSpec((1, _BS, s), lambda b, i: (b, i, 0)),
        out_shape=jax.ShapeDtypeStruct((batch, s, s), jnp.float32),
        compiler_params=pltpu.CompilerParams(
            dimension_semantics=("parallel", "parallel")),
    )


def _rope_body(pos_ref, cos_ref, sin_ref):
    p = pos_ref[:, :].astype(jnp.float32)
    s = p.shape[1]
    k = lax.broadcasted_iota(jnp.int32, (HEAD_DIM, 1), 0)
    k = jnp.where(k >= HEAD_DIM // 2, k - HEAD_DIM // 2, k).astype(jnp.float32)
    inv = jnp.exp(k * jnp.float32(-2.0 * math.log(ROPE_THETA) / HEAD_DIM))
    freqs = inv * p
    cos_ref[:, :] = jnp.cos(freqs)
    sin_ref[:, :] = jnp.sin(freqs)


def _make_rope(s: int):
    return pl.pallas_call(
        _rope_body,
        out_shape=[
            jax.ShapeDtypeStruct((HEAD_DIM, s), jnp.float32),
            jax.ShapeDtypeStruct((HEAD_DIM, s), jnp.float32),
        ],
    )


def kernel(input_ids, attention_mask, position_ids, control_classes, labels, W):
    b, s = input_ids.shape
    ids = input_ids.reshape(-1).astype(jnp.int32)

    mask = _make_mask(b, s)(attention_mask.reshape(b, 1, s)).reshape(b, 1, s, s)

    emb = _make_sc_gather(b * s)(ids, W)
    hidden = emb.reshape(b, s, HIDDEN)

    cos_t, sin_t = _make_rope(s)(position_ids.reshape(1, s))
    cos = cos_t.T.reshape(1, s, HEAD_DIM)
    sin = sin_t.T.reshape(1, s, HEAD_DIM)

    cache_position = jnp.arange(0, s)
    return (hidden, mask, cos, sin, cache_position, control_classes, labels)


# speedup vs baseline: 1.0138x; 1.0064x over previous
import functools
import math

import jax
import jax.numpy as jnp
import numpy as np
from jax import lax
from jax.experimental import pallas as pl
from jax.experimental.pallas import tpu as pltpu
from jax.experimental.pallas import tpu_sc as plsc

HIDDEN = 2048
HEAD_DIM = 128
ROPE_THETA = 10000.0
_NORM = np.float32(float(HIDDEN) ** 0.5)
_F32_MIN = float(np.finfo(np.float32).min)

_NC = 2
_NS = 16
_NW = _NC * _NS


def _make_sc_gather(n_tokens: int):
    per_w = n_tokens // _NW
    C = 8
    NIN = 2
    NOUT = 2
    nchunks = per_w // C
    mesh = plsc.VectorSubcoreMesh(core_axis_name="c", subcore_axis_name="s")

    @functools.partial(
        pl.kernel,
        mesh=mesh,
        out_type=jax.ShapeDtypeStruct((n_tokens, HIDDEN), jnp.float32),
        scratch_types=(
            [pltpu.VMEM((per_w,), jnp.int32)]
            + [pltpu.VMEM((C, HIDDEN), jnp.float32)] * (NIN + NOUT)
            + [pltpu.SemaphoreType.DMA] * (NIN + NOUT)
        ),
    )
    def sc_gather(ids_hbm, w_hbm, out_hbm, idx_v, *bufs):
        inb = list(bufs[:NIN])
        outb = list(bufs[NIN:NIN + NOUT])
        sin = list(bufs[NIN + NOUT:2 * NIN + NOUT])
        son = list(bufs[2 * NIN + NOUT:])
        wid = lax.axis_index("s") * _NC + lax.axis_index("c")
        base = wid * per_w
        pltpu.sync_copy(ids_hbm.at[pl.ds(base, per_w)], idx_v)

        def gather_copy(c, b):
            return pltpu.make_async_copy(
                w_hbm.at[idx_v.at[pl.ds(c * C, C)]], inb[b], sin[b])

        def out_copy(c, b):
            return pltpu.make_async_copy(
                outb[b], out_hbm.at[pl.ds(base + c * C, C)], son[b])

        for b in range(NIN):
            gather_copy(b, b).start()

        def body(c0):
            for b in range(NIN):
                c = c0 + b
                bo = b % NOUT
                gather_copy(c, b).wait()

                @pl.when(c >= NOUT)
                def _():
                    out_copy(c, bo).wait()

                def scale_row(r, carry):
                    for k in range(HIDDEN // 16):
                        outb[bo][r, pl.ds(k * 16, 16)] = (
                            inb[b][r, pl.ds(k * 16, 16)] * _NORM)
                    return carry

                lax.fori_loop(0, C, scale_row, 0)

                @pl.when(c + NIN < nchunks)
                def _():
                    gather_copy(c + NIN, b).start()

                out_copy(c, bo).start()

        pl.loop(0, nchunks, step=NIN)(body)
        for b in range(NOUT):
            out_copy(nchunks - NOUT + b, b).wait()

    return sc_gather


_BS = 512


def _mask_body(am_ref, out_ref):
    i = pl.program_id(1)
    bs, s = out_ref.shape[1], out_ref.shape[2]
    rows = lax.broadcasted_iota(jnp.int32, (bs, s), 0) + i * bs
    cols = lax.broadcasted_iota(jnp.int32, (bs, s), 1)
    masked = (cols > rows) | (am_ref[0] == 0)
    out_ref[0] = jnp.where(masked, jnp.float32(_F32_MIN), jnp.float32(0.0))


def _make_mask(batch: int, s: int):
    return pl.pallas_call(
        _mask_body,
        grid=(batch, s // _BS),
        in_specs=[pl.BlockSpec((1, 1, s), lambda b, i: (b, 0, 0))],
        out_specs=pl.BlockSpec((1, _BS, s), lambda b, i: (b, i, 0)),
        out_shape=jax.ShapeDtypeStruct((batch, s, s), jnp.float32),
    )


def _rope_body(pos_ref, cos_ref, sin_ref):
    p = pos_ref[:, :].astype(jnp.float32)
    k = lax.broadcasted_iota(jnp.int32, (1, HEAD_DIM // 2), 1).astype(jnp.float32)
    inv = jnp.exp(k * jnp.float32(-2.0 * math.log(ROPE_THETA) / HEAD_DIM))
    freqs = p * inv
    emb = jnp.concatenate([freqs, freqs], axis=-1)
    cos_ref[:, :] = jnp.cos(emb)
    sin_ref[:, :] = jnp.sin(emb)


def _make_rope(s: int):
    return pl.pallas_call(
        _rope_body,
        out_shape=[
            jax.ShapeDtypeStruct((s, HEAD_DIM), jnp.float32),
            jax.ShapeDtypeStruct((s, HEAD_DIM), jnp.float32),
        ],
    )


def kernel(input_ids, attention_mask, position_ids, control_classes, labels, W):
    b, s = input_ids.shape
    ids = input_ids.reshape(-1).astype(jnp.int32)

    mask = _make_mask(b, s)(attention_mask.reshape(b, 1, s)).reshape(b, 1, s, s)

    emb = _make_sc_gather(b * s)(ids, W)
    hidden = emb.reshape(b, s, HIDDEN)

    cos2, sin2 = _make_rope(s)(position_ids.reshape(s, 1))
    cos = cos2.reshape(1, s, HEAD_DIM)
    sin = sin2.reshape(1, s, HEAD_DIM)

    cache_position = jnp.arange(0, s)
    return (hidden, mask, cos, sin, cache_position, control_classes, labels)
